# baseline (speedup 1.0000x reference)
import jax
import jax.numpy as jnp
from jax.experimental import pallas as pl
from jax.experimental.pallas import tpu as pltpu

_BT = 2


def _one_batch(p, sup, prior, ms_ref, pe_ref, mi_ref, bb):
    k = p.shape[-1]
    pt = jax.lax.transpose(p, (1, 0))
    logp = jnp.log(jnp.maximum(pt, 1e-12))
    ent = -jnp.sum(pt * logp, axis=0, keepdims=True)

    m = jnp.max(pt, axis=0, keepdims=True)
    iota = jax.lax.broadcasted_iota(jnp.int32, pt.shape, 0)
    idx = jnp.min(jnp.where(pt == m, iota, k), axis=0, keepdims=True)
    supt = jax.lax.transpose(sup, (1, 0))
    ms = jnp.sum(jnp.where(iota == idx, supt, 0.0), axis=0, keepdims=True)

    prior_ent = -jnp.sum(prior * jnp.log(jnp.maximum(prior, 1e-12)))
    mi = jnp.maximum(prior_ent - ent, 0.0)

    ms_ref[bb] = ms
    pe_ref[bb] = ent
    mi_ref[bb] = mi


def _fused_body(pa_ref, pb_ref, sup_ref, prior_ref,
                msa_ref, pea_ref, mia_ref, msb_ref, peb_ref, mib_ref):
    half = sup_ref.shape[0] // 2
    for bb in range(_BT):
        _one_batch(pa_ref[bb], sup_ref[bb], prior_ref[bb],
                   msa_ref, pea_ref, mia_ref, bb)
    for bb in range(_BT):
        _one_batch(pb_ref[bb], sup_ref[half + bb], prior_ref[half + bb],
                   msb_ref, peb_ref, mib_ref, bb)


def kernel(posterior_probabilities, support, prior_probabilities):
    b, g, k = posterior_probabilities.shape
    nb = b // (2 * _BT)
    hb = b // 2

    sup3 = support.reshape(b, 1, k)
    prior3 = prior_probabilities.reshape(b, 1, k)

    out_sd = jax.ShapeDtypeStruct((hb, 1, g), jnp.float32)
    out_spec = pl.BlockSpec((_BT, 1, g), lambda i: (i, 0, 0))
    sp_spec = pl.BlockSpec((2 * _BT, 1, k), lambda i: (i, 0, 0))
    outs = pl.pallas_call(
        _fused_body,
        grid=(nb,),
        in_specs=[
            pl.BlockSpec((_BT, g, k), lambda i: (i, 0, 0)),
            pl.BlockSpec((_BT, g, k), lambda i: (i + nb, 0, 0)),
            sp_spec,
            sp_spec,
        ],
        out_specs=[out_spec] * 6,
        out_shape=[out_sd] * 6,
        compiler_params=pltpu.CompilerParams(
            dimension_semantics=("parallel",),
        ),
    )(posterior_probabilities, posterior_probabilities,
      _interleave(sup3, hb), _interleave(prior3, hb))

    msa, pea, mia, msb, peb, mib = outs
    ms = jnp.concatenate([msa, msb], axis=0).reshape(b, g)
    pe = jnp.concatenate([pea, peb], axis=0).reshape(b, g)
    mi = jnp.concatenate([mia, mib], axis=0).reshape(b, g)
    return (ms, pe, mi)


def _interleave(x, hb):
    xa = x[:hb].reshape(-1, _BT, *x.shape[1:])
    xb = x[hb:].reshape(-1, _BT, *x.shape[1:])
    return jnp.concatenate([xa, xb], axis=1).reshape(-1, *x.shape[1:])


# speedup vs baseline: 1.0666x; 1.0666x over previous
import jax
import jax.numpy as jnp
from jax.experimental import pallas as pl
from jax.experimental.pallas import tpu as pltpu

_BT = 4


def _fused_body(post_ref, sup_ref, prior_ref, ms_ref, pe_ref, mi_ref):
    k = post_ref.shape[-1]
    for bb in range(_BT):
        p = post_ref[bb]
        sup = sup_ref[bb]
        prior = prior_ref[bb]

        pt = jax.lax.transpose(p, (1, 0))
        logp = jnp.log(jnp.maximum(pt, 1e-12))
        ent = -jnp.sum(pt * logp, axis=0, keepdims=True)

        m = jnp.max(pt, axis=0, keepdims=True)
        iota = jax.lax.broadcasted_iota(jnp.int32, pt.shape, 0)
        idx = jnp.min(jnp.where(pt == m, iota, k), axis=0, keepdims=True)
        supt = jax.lax.transpose(sup, (1, 0))
        ms = jnp.sum(jnp.where(iota == idx, supt, 0.0), axis=0, keepdims=True)

        prior_ent = -jnp.sum(prior * jnp.log(jnp.maximum(prior, 1e-12)))
        mi = jnp.maximum(prior_ent - ent, 0.0)

        ms_ref[bb] = ms
        pe_ref[bb] = ent
        mi_ref[bb] = mi


def kernel(posterior_probabilities, support, prior_probabilities):
    b, g, k = posterior_probabilities.shape
    nb = b // _BT

    sup3 = support.reshape(b, 1, k)
    prior3 = prior_probabilities.reshape(b, 1, k)

    out_sd = jax.ShapeDtypeStruct((b, 1, g), jnp.float32)
    out_spec = pl.BlockSpec((_BT, 1, g), lambda i: (i, 0, 0))
    ms, pe, mi = pl.pallas_call(
        _fused_body,
        grid=(nb,),
        in_specs=[
            pl.BlockSpec((_BT, g, k), lambda i: (i, 0, 0)),
            pl.BlockSpec((_BT, 1, k), lambda i: (i, 0, 0)),
            pl.BlockSpec((_BT, 1, k), lambda i: (i, 0, 0)),
        ],
        out_specs=[out_spec, out_spec, out_spec],
        out_shape=[out_sd, out_sd, out_sd],
        compiler_params=pltpu.CompilerParams(
            dimension_semantics=("parallel",),
        ),
    )(posterior_probabilities, sup3, prior3)

    return (ms.reshape(b, g), pe.reshape(b, g), mi.reshape(b, g))
